# baseline (speedup 1.0000x reference)
import jax
import jax.numpy as jnp
from jax.experimental import pallas as pl

N = 32768
D = 64
FF = 48
H = 3 * FF

BN2 = 2048


def _moe_block(x_ref, w1a_ref, b1a_ref, w1b_ref, b1b_ref, w1g_ref, bg_ref,
               w2_ref, sb2_ref, rb2_ref, out_ref):
    x = x_ref[...]

    ha = jnp.dot(x, w1a_ref[...], preferred_element_type=jnp.float32) + b1a_ref[...]
    hb = jnp.dot(x, w1b_ref[...], preferred_element_type=jnp.float32) + b1b_ref[...]
    act = (ha * jax.nn.sigmoid(ha)) * hb

    lg = jnp.dot(x, w1g_ref[...], preferred_element_type=jnp.float32) + bg_ref[...]
    m0 = (lg[:, 1:2] > lg[:, 0:1]).astype(jnp.float32)
    m1 = (lg[:, 3:4] > lg[:, 2:3]).astype(jnp.float32)
    w0 = jax.nn.sigmoid(jnp.abs(lg[:, 1:2] - lg[:, 0:1]))
    w1 = jax.nn.sigmoid(jnp.abs(lg[:, 3:4] - lg[:, 2:3]))

    col = jax.lax.broadcasted_iota(jnp.int32, (1, 2 * H), 1)
    one = jnp.float32(1.0)
    scale = jnp.where(
        col < FF, one,
        jnp.where(col < 2 * FF, w0 * (one - m0),
                  jnp.where(col < H, w0 * m0,
                            jnp.where(col < H + FF, one,
                                      jnp.where(col < H + 2 * FF,
                                                w1 * (one - m1), w1 * m1)))))
    act = act * scale

    out = jnp.dot(act, w2_ref[...], preferred_element_type=jnp.float32)

    rb2 = rb2_ref[...]
    left = (w0 * (one - m0)) * rb2[0:1] + (w0 * m0) * rb2[1:2]
    right = (w1 * (one - m1)) * rb2[0:1] + (w1 * m1) * rb2[1:2]
    out_ref[...] = out + sb2_ref[...] + jnp.concatenate([left, right], axis=1)


@jax.jit
def kernel(x, sw1, sb1, sw2, sb2, rw1, rb1, rw2, rb2, gw, gb):
    w1a = jnp.concatenate([sw1[:, :FF], rw1[0][:, :FF], rw1[1][:, :FF]], axis=1)
    w1b = jnp.concatenate([sw1[:, FF:], rw1[0][:, FF:], rw1[1][:, FF:]], axis=1)
    b1a = jnp.concatenate([sb1[:FF], rb1[0][:FF], rb1[1][:FF]], axis=0)
    b1b = jnp.concatenate([sb1[FF:], rb1[0][FF:], rb1[1][FF:]], axis=0)
    z = jnp.zeros((D, H), jnp.float32)
    w1a2 = jnp.concatenate(
        [jnp.concatenate([w1a, z], axis=1), jnp.concatenate([z, w1a], axis=1)], axis=0)
    w1b2 = jnp.concatenate(
        [jnp.concatenate([w1b, z], axis=1), jnp.concatenate([z, w1b], axis=1)], axis=0)
    b1a2 = jnp.concatenate([b1a, b1a], axis=0)[None, :]
    b1b2 = jnp.concatenate([b1b, b1b], axis=0)[None, :]
    zg = jnp.zeros((D, 2), jnp.float32)
    w1g2 = jnp.concatenate(
        [jnp.concatenate([gw, zg], axis=1), jnp.concatenate([zg, gw], axis=1)], axis=0)
    bg2 = jnp.concatenate([gb, gb], axis=0)[None, :]
    w2 = jnp.concatenate([sw2, rw2[0], rw2[1]], axis=0)
    z2 = jnp.zeros((H, D), jnp.float32)
    w22 = jnp.concatenate(
        [jnp.concatenate([w2, z2], axis=1), jnp.concatenate([z2, w2], axis=1)], axis=0)
    sb22 = jnp.concatenate([sb2, sb2], axis=0)[None, :]

    x2 = x.reshape(N // 2, 2 * D)
    grid = ((N // 2) // BN2,)
    full = lambda *s: pl.BlockSpec(s, lambda i: (0,) * len(s))
    out2 = pl.pallas_call(
        _moe_block,
        grid=grid,
        in_specs=[
            pl.BlockSpec((BN2, 2 * D), lambda i: (i, 0)),
            full(2 * D, 2 * H), full(1, 2 * H),
            full(2 * D, 2 * H), full(1, 2 * H),
            full(2 * D, 4), full(1, 4),
            full(2 * H, 2 * D), full(1, 2 * D), full(2, D),
        ],
        out_specs=pl.BlockSpec((BN2, 2 * D), lambda i: (i, 0)),
        out_shape=jax.ShapeDtypeStruct((N // 2, 2 * D), jnp.float32),
    )(x2, w1a2, b1a2, w1b2, b1b2, w1g2, bg2, w22, sb22, rb2)
    return out2.reshape(N, D)


# speedup vs baseline: 1.4247x; 1.4247x over previous
import jax
import jax.numpy as jnp
import numpy as np
from jax.experimental import pallas as pl

N = 32768
D = 64
FF = 48
C = 3 * FF + 3

BN = 4096


def _moe_block(x_ref, w1a_ref, b1a_ref, w1b_ref, b1b_ref, tvec_ref,
               ind_ref, w2_ref, out_ref):
    x = x_ref[...]

    ha = jnp.dot(x, w1a_ref[...], preferred_element_type=jnp.float32) + b1a_ref[...]
    hb = jnp.dot(x, w1b_ref[...], preferred_element_type=jnp.float32) + b1b_ref[...]

    sig = jax.nn.sigmoid(ha)
    col = jax.lax.broadcasted_iota(jnp.int32, (1, C), 1)
    mask = jnp.where(sig > tvec_ref[...], 1.0, 0.0)
    g = jnp.where(col < 3 * FF, ha, mask)
    act = (sig * g) * hb

    scale = jnp.dot(act, ind_ref[...], preferred_element_type=jnp.float32)
    act = act * scale

    out_ref[...] = jnp.dot(act, w2_ref[...], preferred_element_type=jnp.float32)


@jax.jit
def kernel(x, sw1, sb1, sw2, sb2, rw1, rb1, rw2, rb2, gw, gb):
    f32 = jnp.float32
    gd = gw[:, 1] - gw[:, 0]
    w1a = jnp.concatenate(
        [sw1[:, :FF], rw1[0][:, :FF], rw1[1][:, :FF],
         gd[:, None], -gd[:, None], jnp.zeros((D, 1), f32)], axis=1)
    gbd = gb[1] - gb[0]
    b1a = jnp.concatenate(
        [sb1[:FF], rb1[0][:FF], rb1[1][:FF],
         jnp.stack([gbd, -gbd, f32(60.0)])], axis=0)[None, :]
    z1 = jnp.zeros((D, 3), f32)
    w1b = jnp.concatenate(
        [sw1[:, FF:], rw1[0][:, FF:], rw1[1][:, FF:], z1], axis=1)
    b1b = jnp.concatenate(
        [sb1[FF:], rb1[0][FF:], rb1[1][FF:], jnp.ones((3,), f32)], axis=0)[None, :]

    tv = np.zeros((1, C), np.float32)
    tv[0, 3 * FF] = 0.5
    tv[0, 3 * FF + 1] = np.nextafter(np.float32(0.5), np.float32(0.0))
    tvec = jnp.asarray(tv)

    ind = np.zeros((C, C), np.float32)
    ind[3 * FF, 2 * FF:3 * FF] = 1.0
    ind[3 * FF + 1, FF:2 * FF] = 1.0
    ind[3 * FF + 2, :FF] = 1.0
    ind[3 * FF + 2, 3 * FF:] = 1.0
    ind = jnp.asarray(ind)

    w2 = jnp.concatenate(
        [sw2, rw2[0], rw2[1], rb2[1][None, :], rb2[0][None, :], sb2[None, :]],
        axis=0)

    grid = (N // BN,)
    full = lambda *s: pl.BlockSpec(s, lambda i: (0,) * len(s))
    return pl.pallas_call(
        _moe_block,
        grid=grid,
        in_specs=[
            pl.BlockSpec((BN, D), lambda i: (i, 0)),
            full(D, C), full(1, C), full(D, C), full(1, C), full(1, C),
            full(C, C), full(C, D),
        ],
        out_specs=pl.BlockSpec((BN, D), lambda i: (i, 0)),
        out_shape=jax.ShapeDtypeStruct((N, D), jnp.float32),
    )(x, w1a, b1a, w1b, b1b, tvec, ind, w2)


# speedup vs baseline: 1.4649x; 1.0283x over previous
import jax
import jax.numpy as jnp
from jax.experimental import pallas as pl

N = 32768
D = 64
FF = 48

BN = 4096


def _moe_block(x_ref, sw1x_ref, sb1x_ref, sw2_ref, sb2_ref,
               rw1_ref, rb1_ref, rw2_ref, rb2_ref, out_ref):
    x = x_ref[...]

    hs = jnp.dot(x, sw1x_ref[...], preferred_element_type=jnp.float32) + sb1x_ref[...]
    a = hs[:, :FF]
    b = hs[:, FF:2 * FF]
    l0 = hs[:, 2 * FF:2 * FF + 1]
    l1 = hs[:, 2 * FF + 1:2 * FF + 2]
    shared = jnp.dot((a * jax.nn.sigmoid(a)) * b, sw2_ref[...],
                     preferred_element_type=jnp.float32) + sb2_ref[...]

    def routed_expert(e):
        h = jnp.dot(x, rw1_ref[e], preferred_element_type=jnp.float32) + rb1_ref[e]
        a = h[:, :FF]
        b = h[:, FF:]
        return jnp.dot((a * jax.nn.sigmoid(a)) * b, rw2_ref[e],
                       preferred_element_type=jnp.float32) + rb2_ref[e]

    o0 = routed_expert(0)
    o1 = routed_expert(1)

    pick1 = l1 > l0
    w = jax.nn.sigmoid(jnp.abs(l1 - l0))
    routed = jnp.where(pick1, o1, o0) * w
    out_ref[...] = shared + routed


@jax.jit
def kernel(x, sw1, sb1, sw2, sb2, rw1, rb1, rw2, rb2, gw, gb):
    sw1x = jnp.concatenate([sw1, gw], axis=1)
    sb1x = jnp.concatenate([sb1, gb], axis=0)[None]
    grid = (N // BN,)
    full = lambda *s: pl.BlockSpec(s, lambda i: (0,) * len(s))
    return pl.pallas_call(
        _moe_block,
        grid=grid,
        in_specs=[
            pl.BlockSpec((BN, D), lambda i: (i, 0)),
            full(D, 2 * FF + 2), full(1, 2 * FF + 2), full(FF, D), full(D),
            full(2, D, 2 * FF), full(2, 2 * FF), full(2, FF, D), full(2, D),
        ],
        out_specs=pl.BlockSpec((BN, D), lambda i: (i, 0)),
        out_shape=jax.ShapeDtypeStruct((N, D), jnp.float32),
    )(x, sw1x, sb1x, sw2, sb2, rw1, rb1, rw2, rb2)
